# baseline (speedup 1.0000x reference)
import functools

import jax
import jax.numpy as jnp
from jax import lax
from jax.experimental import pallas as pl
from jax.experimental.pallas import tpu as pltpu
from jax.experimental.pallas import tpu_sc as plsc

R = 32
RES = 512
P = 1048576

NC = 2
NS = 16
NW = NC * NS
L = 16

C = 128
PPW = P // NW
NCH = PPW // C


@functools.partial(
    pl.kernel,
    mesh=plsc.VectorSubcoreMesh(core_axis_name="c", subcore_axis_name="s"),
    out_type=jax.ShapeDtypeStruct((P, R), jnp.float32),
    compiler_params=pltpu.CompilerParams(use_tc_tiling_on_sc=False,
                                         needs_layout_passes=False),
    scratch_types=[
        pltpu.VMEM((C, 3), jnp.float32),
        pltpu.VMEM((C, 3), jnp.float32),
        pltpu.VMEM((4, C), jnp.int32),
        pltpu.VMEM((4, C), jnp.int32),
        pltpu.VMEM((4, C), jnp.int32),
        pltpu.VMEM((4, C), jnp.int32),
        pltpu.VMEM((4, C), jnp.int32),
        pltpu.VMEM((4, C), jnp.int32),
        pltpu.VMEM((2, C), jnp.float32),
        pltpu.VMEM((2, C), jnp.float32),
        pltpu.VMEM((2, C), jnp.float32),
        pltpu.VMEM((2, C), jnp.float32),
        pltpu.VMEM((2, C), jnp.float32),
        pltpu.VMEM((2, C), jnp.float32),
        pltpu.VMEM((4 * C, R), jnp.float32),
        pltpu.VMEM((4 * C, R), jnp.float32),
        pltpu.VMEM((4 * C, R), jnp.float32),
        pltpu.VMEM((4 * C, R), jnp.float32),
        pltpu.VMEM((4 * C, R), jnp.float32),
        pltpu.VMEM((4 * C, R), jnp.float32),
        pltpu.VMEM((C, R), jnp.float32),
        pltpu.VMEM((C, R), jnp.float32),
        pltpu.VMEM((48, R), jnp.float32),
        pltpu.SemaphoreType.DMA,
        pltpu.SemaphoreType.DMA,
        pltpu.SemaphoreType.DMA,
        pltpu.SemaphoreType.DMA,
        pltpu.SemaphoreType.DMA,
        pltpu.SemaphoreType.DMA,
    ],
)
def _sc_fused(cs, t0, t1, t2, lt0, lt1, lt2, out,
              c0, c1,
              i00, i01, i02, i10, i11, i12,
              w00, w01, w02, w10, w11, w12,
              g00, g01, g02, g10, g11, g12,
              ov0, ov1, lv,
              csem0, csem1, gsem0, gsem1, osem0, osem1):
    wid = lax.axis_index("s") * NC + lax.axis_index("c")
    wbase = wid * PPW

    tbls = (t0, t1, t2)
    par = (
        dict(c=c0, csem=csem0, idx=(i00, i01, i02), w=(w00, w01, w02),
             g=(g00, g01, g02), gsem=gsem0, ov=ov0, osem=osem0),
        dict(c=c1, csem=csem1, idx=(i10, i11, i12), w=(w10, w11, w12),
             g=(g10, g11, g12), gsem=gsem1, ov=ov1, osem=osem1),
    )

    pltpu.sync_copy(lt0.at[pl.ds(248, 16)], lv.at[pl.ds(0, 16)])
    pltpu.sync_copy(lt1.at[pl.ds(248, 16)], lv.at[pl.ds(16, 16)])
    pltpu.sync_copy(lt2.at[pl.ds(248, 16)], lv.at[pl.ds(32, 16)])
    lc = [[0.5 * lv[16 * i + 7, pl.ds(h, L)] + 0.5 * lv[16 * i + 8, pl.ds(h, L)]
           for h in (0, L)] for i in range(3)]

    def coords_issue(q, b):
        pltpu.async_copy(cs.at[pl.ds(wbase + q * C, C)],
                         par[b]["c"], par[b]["csem"])

    def gathers_issue(b):
        pb = par[b]
        pltpu.make_async_copy(cs.at[pl.ds(0, C)], pb["c"], pb["csem"]).wait()
        cbuf = pb["c"]
        lane = lax.iota(jnp.int32, L)
        zero = jnp.zeros((L,), jnp.int32)

        def grp(gi, cr):
            o = gi * L
            rows = lane + o
            xg = plsc.load_gather(cbuf, [rows, zero])
            yg = plsc.load_gather(cbuf, [rows, zero + 1])
            zg = plsc.load_gather(cbuf, [rows, zero + 2])

            def prep(g):
                t = (g + 1.0) * 0.5 * (RES - 1.0)
                ti = jnp.minimum(t.astype(jnp.int32), RES - 2)
                fr = t - ti.astype(jnp.float32)
                return ti, fr

            xi, xf = prep(xg)
            yi, yf = prep(yg)
            zi, zf = prep(zg)
            for pi, ((gxi, gxf), (gyi, gyf)) in enumerate(
                    (((xi, xf), (yi, yf)),
                     ((xi, xf), (zi, zf)),
                     ((yi, yf), (zi, zf)))):
                idxr = pb["idx"][pi]
                wr = pb["w"][pi]
                base = gyi * RES + gxi
                idxr[0, pl.ds(o, L)] = base
                idxr[1, pl.ds(o, L)] = base + 1
                idxr[2, pl.ds(o, L)] = base + RES
                idxr[3, pl.ds(o, L)] = base + (RES + 1)
                wr[0, pl.ds(o, L)] = gxf
                wr[1, pl.ds(o, L)] = gyf
            return cr

        lax.fori_loop(0, C // L, grp, 0)
        for pi in range(3):
            for k in range(4):
                pltpu.async_copy(tbls[pi].at[pb["idx"][pi].at[k]],
                                 pb["g"][pi].at[pl.ds(k * C, C)], pb["gsem"])

    def combine(q, b, drain_out):
        pb = par[b]
        for pi in range(3):
            pltpu.make_async_copy(t0.at[pl.ds(0, 4 * C)], pb["g"][pi],
                                  pb["gsem"]).wait()
        ovb = pb["ov"]

        def grp(gi, cr):
            ob = gi * L
            wvec = [[pb["w"][pi][k, pl.ds(ob, L)] for k in range(2)]
                    for pi in range(3)]
            for j in range(L):
                p = ob + j
                acc_a = acc_b = None
                for pi in range(3):
                    gr = pb["g"][pi]
                    wx = wvec[pi][0][j]
                    wy = wvec[pi][1][j]
                    v00a = gr[p, pl.ds(0, L)]
                    v01a = gr[C + p, pl.ds(0, L)]
                    v10a = gr[2 * C + p, pl.ds(0, L)]
                    v11a = gr[3 * C + p, pl.ds(0, L)]
                    t0a = v00a + wx * (v01a - v00a)
                    t1a = v10a + wx * (v11a - v10a)
                    sa = t0a + wy * (t1a - t0a)
                    v00b = gr[p, pl.ds(L, L)]
                    v01b = gr[C + p, pl.ds(L, L)]
                    v10b = gr[2 * C + p, pl.ds(L, L)]
                    v11b = gr[3 * C + p, pl.ds(L, L)]
                    t0b = v00b + wx * (v01b - v00b)
                    t1b = v10b + wx * (v11b - v10b)
                    sb = t0b + wy * (t1b - t0b)
                    if pi == 0:
                        acc_a = sa * lc[0][0]
                        acc_b = sb * lc[0][1]
                    else:
                        acc_a = acc_a + sa * lc[pi][0]
                        acc_b = acc_b + sb * lc[pi][1]
                ovb[p, pl.ds(0, L)] = acc_a
                ovb[p, pl.ds(L, L)] = acc_b
            return cr

        @pl.when(drain_out)
        def _():
            pltpu.make_async_copy(ovb, out.at[pl.ds(0, C)], pb["osem"]).wait()

        lax.fori_loop(0, C // L, grp, 0)
        pltpu.async_copy(ovb, out.at[pl.ds(wbase + q * C, C)], pb["osem"])

    coords_issue(0, 0)
    coords_issue(1, 1)
    gathers_issue(0)
    coords_issue(2, 0)
    gathers_issue(1)
    coords_issue(3, 1)

    def body(t, carry):
        q = 2 * t
        combine(q, 0, t > 0)
        gathers_issue(0)
        @pl.when(t < NCH // 2 - 2)
        def _():
            coords_issue(q + 4, 0)
        combine(q + 1, 1, t > 0)
        gathers_issue(1)
        @pl.when(t < NCH // 2 - 2)
        def _():
            coords_issue(q + 5, 1)
        return carry

    lax.fori_loop(0, NCH // 2 - 1, body, 0)

    combine(NCH - 2, 0, True)
    combine(NCH - 1, 1, True)
    pltpu.make_async_copy(ov0, out.at[pl.ds(0, C)], osem0).wait()
    pltpu.make_async_copy(ov1, out.at[pl.ds(0, C)], osem1).wait()


def kernel(positions, plane0, plane1, plane2, line0, line1, line2):
    original_shape = positions.shape[:-1]
    cs = positions.reshape(-1, 3)
    t0 = plane0[0].transpose(1, 2, 0).reshape(RES * RES, R)
    t1 = plane1[0].transpose(1, 2, 0).reshape(RES * RES, R)
    t2 = plane2[0].transpose(1, 2, 0).reshape(RES * RES, R)
    lt0 = line0[0, :, :, 0].T
    lt1 = line1[0, :, :, 0].T
    lt2 = line2[0, :, :, 0].T
    out = _sc_fused(cs, t0, t1, t2, lt0, lt1, lt2)
    return out.reshape(*original_shape, R)


# speedup vs baseline: 2.0832x; 2.0832x over previous
import functools

import jax
import jax.numpy as jnp
from jax import lax
from jax.experimental import pallas as pl
from jax.experimental.pallas import tpu as pltpu
from jax.experimental.pallas import tpu_sc as plsc

R = 32
RES = 512
P = 1048576

NC = 2
NS = 16
NW = NC * NS
L = 16

C = 128
PPW = P // NW
NCH = PPW // C


@functools.partial(
    pl.kernel,
    mesh=plsc.VectorSubcoreMesh(core_axis_name="c", subcore_axis_name="s"),
    out_type=jax.ShapeDtypeStruct((P, R), jnp.float32),
    compiler_params=pltpu.CompilerParams(use_tc_tiling_on_sc=False,
                                         needs_layout_passes=False),
    scratch_types=[
        pltpu.VMEM((3, C), jnp.float32),
        pltpu.VMEM((3, C), jnp.float32),
        pltpu.VMEM((4, C), jnp.int32),
        pltpu.VMEM((4, C), jnp.int32),
        pltpu.VMEM((4, C), jnp.int32),
        pltpu.VMEM((4, C), jnp.int32),
        pltpu.VMEM((4, C), jnp.int32),
        pltpu.VMEM((4, C), jnp.int32),
        pltpu.VMEM((2, C), jnp.float32),
        pltpu.VMEM((2, C), jnp.float32),
        pltpu.VMEM((2, C), jnp.float32),
        pltpu.VMEM((2, C), jnp.float32),
        pltpu.VMEM((2, C), jnp.float32),
        pltpu.VMEM((2, C), jnp.float32),
        pltpu.VMEM((4 * C, R), jnp.float32),
        pltpu.VMEM((4 * C, R), jnp.float32),
        pltpu.VMEM((4 * C, R), jnp.float32),
        pltpu.VMEM((4 * C, R), jnp.float32),
        pltpu.VMEM((4 * C, R), jnp.float32),
        pltpu.VMEM((4 * C, R), jnp.float32),
        pltpu.VMEM((C, R), jnp.float32),
        pltpu.VMEM((C, R), jnp.float32),
        pltpu.VMEM((48, R), jnp.float32),
        pltpu.SemaphoreType.DMA,
        pltpu.SemaphoreType.DMA,
        pltpu.SemaphoreType.DMA,
        pltpu.SemaphoreType.DMA,
        pltpu.SemaphoreType.DMA,
        pltpu.SemaphoreType.DMA,
    ],
)
def _sc_fused(cs, t0, t1, t2, lt0, lt1, lt2, out,
              c0, c1,
              i00, i01, i02, i10, i11, i12,
              w00, w01, w02, w10, w11, w12,
              g00, g01, g02, g10, g11, g12,
              ov0, ov1, lv,
              csem0, csem1, gsem0, gsem1, osem0, osem1):
    wid = lax.axis_index("s") * NC + lax.axis_index("c")
    wbase = wid * PPW

    tbls = (t0, t1, t2)
    par = (
        dict(c=c0, csem=csem0, idx=(i00, i01, i02), w=(w00, w01, w02),
             g=(g00, g01, g02), gsem=gsem0, ov=ov0, osem=osem0),
        dict(c=c1, csem=csem1, idx=(i10, i11, i12), w=(w10, w11, w12),
             g=(g10, g11, g12), gsem=gsem1, ov=ov1, osem=osem1),
    )

    pltpu.sync_copy(lt0.at[pl.ds(248, 16)], lv.at[pl.ds(0, 16)])
    pltpu.sync_copy(lt1.at[pl.ds(248, 16)], lv.at[pl.ds(16, 16)])
    pltpu.sync_copy(lt2.at[pl.ds(248, 16)], lv.at[pl.ds(32, 16)])
    lc = [[0.5 * lv[16 * i + 7, pl.ds(h, L)] + 0.5 * lv[16 * i + 8, pl.ds(h, L)]
           for h in (0, L)] for i in range(3)]

    def coords_issue(q, b):
        pltpu.async_copy(cs.at[:, pl.ds(wbase + q * C, C)],
                         par[b]["c"], par[b]["csem"])

    def gathers_issue(b):
        pb = par[b]
        pltpu.make_async_copy(cs.at[:, pl.ds(0, C)], pb["c"], pb["csem"]).wait()
        cbuf = pb["c"]

        def grp(gi, cr):
            o = gi * L
            xg = cbuf[0, pl.ds(o, L)]
            yg = cbuf[1, pl.ds(o, L)]
            zg = cbuf[2, pl.ds(o, L)]

            def prep(g):
                t = (g + 1.0) * 0.5 * (RES - 1.0)
                ti = jnp.minimum(t.astype(jnp.int32), RES - 2)
                fr = t - ti.astype(jnp.float32)
                return ti, fr

            xi, xf = prep(xg)
            yi, yf = prep(yg)
            zi, zf = prep(zg)
            for pi, ((gxi, gxf), (gyi, gyf)) in enumerate(
                    (((xi, xf), (yi, yf)),
                     ((xi, xf), (zi, zf)),
                     ((yi, yf), (zi, zf)))):
                idxr = pb["idx"][pi]
                wr = pb["w"][pi]
                base = gyi * RES + gxi
                idxr[0, pl.ds(o, L)] = base
                idxr[1, pl.ds(o, L)] = base + 1
                idxr[2, pl.ds(o, L)] = base + RES
                idxr[3, pl.ds(o, L)] = base + (RES + 1)
                wr[0, pl.ds(o, L)] = gxf
                wr[1, pl.ds(o, L)] = gyf
            return cr

        lax.fori_loop(0, C // L, grp, 0)
        for pi in range(3):
            for k in range(4):
                pltpu.async_copy(tbls[pi].at[pb["idx"][pi].at[k]],
                                 pb["g"][pi].at[pl.ds(k * C, C)], pb["gsem"])

    def combine(q, b, drain_out):
        pb = par[b]
        for pi in range(3):
            pltpu.make_async_copy(t0.at[pl.ds(0, 4 * C)], pb["g"][pi],
                                  pb["gsem"]).wait()
        ovb = pb["ov"]

        def grp(gi, cr):
            ob = gi * L
            wvec = [[pb["w"][pi][k, pl.ds(ob, L)] for k in range(2)]
                    for pi in range(3)]
            for j in range(L):
                p = ob + j
                acc_a = acc_b = None
                for pi in range(3):
                    gr = pb["g"][pi]
                    wx = wvec[pi][0][j]
                    wy = wvec[pi][1][j]
                    v00a = gr[p, pl.ds(0, L)]
                    v01a = gr[C + p, pl.ds(0, L)]
                    v10a = gr[2 * C + p, pl.ds(0, L)]
                    v11a = gr[3 * C + p, pl.ds(0, L)]
                    t0a = v00a + wx * (v01a - v00a)
                    t1a = v10a + wx * (v11a - v10a)
                    sa = t0a + wy * (t1a - t0a)
                    v00b = gr[p, pl.ds(L, L)]
                    v01b = gr[C + p, pl.ds(L, L)]
                    v10b = gr[2 * C + p, pl.ds(L, L)]
                    v11b = gr[3 * C + p, pl.ds(L, L)]
                    t0b = v00b + wx * (v01b - v00b)
                    t1b = v10b + wx * (v11b - v10b)
                    sb = t0b + wy * (t1b - t0b)
                    if pi == 0:
                        acc_a = sa * lc[0][0]
                        acc_b = sb * lc[0][1]
                    else:
                        acc_a = acc_a + sa * lc[pi][0]
                        acc_b = acc_b + sb * lc[pi][1]
                ovb[p, pl.ds(0, L)] = acc_a
                ovb[p, pl.ds(L, L)] = acc_b
            return cr

        @pl.when(drain_out)
        def _():
            pltpu.make_async_copy(ovb, out.at[pl.ds(0, C)], pb["osem"]).wait()

        lax.fori_loop(0, C // L, grp, 0)
        pltpu.async_copy(ovb, out.at[pl.ds(wbase + q * C, C)], pb["osem"])

    coords_issue(0, 0)
    coords_issue(1, 1)
    gathers_issue(0)
    coords_issue(2, 0)
    gathers_issue(1)
    coords_issue(3, 1)

    def body(t, carry):
        q = 2 * t
        combine(q, 0, t > 0)
        gathers_issue(0)
        @pl.when(t < NCH // 2 - 2)
        def _():
            coords_issue(q + 4, 0)
        combine(q + 1, 1, t > 0)
        gathers_issue(1)
        @pl.when(t < NCH // 2 - 2)
        def _():
            coords_issue(q + 5, 1)
        return carry

    lax.fori_loop(0, NCH // 2 - 1, body, 0)

    combine(NCH - 2, 0, True)
    combine(NCH - 1, 1, True)
    pltpu.make_async_copy(ov0, out.at[pl.ds(0, C)], osem0).wait()
    pltpu.make_async_copy(ov1, out.at[pl.ds(0, C)], osem1).wait()


def kernel(positions, plane0, plane1, plane2, line0, line1, line2):
    original_shape = positions.shape[:-1]
    cs = positions.reshape(-1, 3).T
    t0 = plane0[0].transpose(1, 2, 0).reshape(RES * RES, R)
    t1 = plane1[0].transpose(1, 2, 0).reshape(RES * RES, R)
    t2 = plane2[0].transpose(1, 2, 0).reshape(RES * RES, R)
    lt0 = line0[0, :, :, 0].T
    lt1 = line1[0, :, :, 0].T
    lt2 = line2[0, :, :, 0].T
    out = _sc_fused(cs, t0, t1, t2, lt0, lt1, lt2)
    return out.reshape(*original_shape, R)
